# baseline (speedup 1.0000x reference)
import jax
import jax.numpy as jnp
from jax import lax
from jax.experimental import pallas as pl
from jax.experimental.pallas import tpu as pltpu
from jax.experimental.pallas import tpu_sc as plsc

B, S, H = 4, 2048, 768
V, T = 100000, 2
N = B * S
NC, NS, L = 2, 16, 16
NW = NC * NS
PB = 64
PH = 8
CH = PH * B
NCHUNK = PB // PH
NLG = H // L


def _emb_body(ids_hbm, tt_hbm, word_hbm, type_hbm, pos_hbm, out_hbm,
              idtmp, tttmp, wbuf0, wbuf1, pbuf0, pbuf1,
              t0v, t1v, dvv, isem, psem, gs0, gs1, os0, os1):
    cid = lax.axis_index("c")
    sid = lax.axis_index("s")
    wid = sid * NC + cid
    pbase = wid * PB

    id_handles = []
    for b in range(B):
        id_handles.append(pltpu.async_copy(
            ids_hbm.at[pl.ds(b * S + pbase, PB)], idtmp.at[b], isem))
    aux_handles = []
    for b in range(B):
        aux_handles.append(pltpu.async_copy(
            tt_hbm.at[pl.ds(b * S + pbase, PB)], tttmp.at[b, pl.ds(0, PB)],
            psem))
    aux_handles.append(pltpu.async_copy(type_hbm.at[0], t0v, psem))
    aux_handles.append(pltpu.async_copy(type_hbm.at[1], t1v, psem))
    for cp in id_handles:
        cp.wait()

    wbuf = [wbuf0, wbuf1]
    pbuf = [pbuf0, pbuf1]
    gsem = [gs0, gs1]
    osem = [os0, os1]

    def issue_chunk(h, p):
        hs = []
        for b in range(B):
            hs.append(pltpu.async_copy(
                word_hbm.at[idtmp.at[b, pl.ds(h * PH, PH)]],
                wbuf[p].at[pl.ds(b * PH, PH)], gsem[p]))
        hs.append(pltpu.async_copy(
            pos_hbm.at[pl.ds(pbase + h * PH, PH)], pbuf[p], gsem[p]))
        return hs

    ghandles = [issue_chunk(0, 0), issue_chunk(1, 1)]
    out_handles = [None, None]

    for cp in aux_handles:
        cp.wait()
    for l in range(NLG):
        sl = pl.ds(l * L, L)
        dvv[sl] = t1v[sl] - t0v[sl]

    for h in range(NCHUNK):
        p = h & 1
        if 1 <= h and h + 1 < NCHUNK:
            if out_handles[1 - p] is not None:
                for oh in out_handles[1 - p]:
                    oh.wait()
            ghandles[1 - p] = issue_chunk(h + 1, 1 - p)
        for gh in ghandles[p]:
            gh.wait()

        wb = wbuf[p]
        pb = pbuf[p]
        ttf = [tttmp[b, pl.ds(h * PH, L)].astype(jnp.float32)
               for b in range(B)]

        def l_body(l, carry, wb=wb, pb=pb, ttf=ttf):
            sl = pl.ds(l * L, L)
            t0 = t0v[sl]
            dv = dvv[sl]
            for r in range(PH):
                pp = pb[r, sl] + t0
                for b in range(B):
                    t = b * PH + r
                    wb[t, sl] = wb[t, sl] + pp + ttf[b][r] * dv
            return carry

        lax.fori_loop(0, NLG, l_body, 0)

        out_handles[p] = []
        for b in range(B):
            row0 = b * S + pbase + h * PH
            out_handles[p].append(pltpu.async_copy(
                wb.at[pl.ds(b * PH, PH)],
                out_hbm.at[pl.ds(row0, PH)], osem[p]))

    for hs in out_handles:
        if hs is not None:
            for oh in hs:
                oh.wait()


@jax.jit
def _emb_call(ids_flat, tt_flat, word_emb, type_emb, pos_emb):
    mesh = plsc.VectorSubcoreMesh(core_axis_name="c", subcore_axis_name="s")
    fn = pl.kernel(
        _emb_body,
        out_type=jax.ShapeDtypeStruct((N, H), jnp.float32),
        mesh=mesh,
        scratch_types=[
            pltpu.VMEM((B, PB), jnp.int32),
            pltpu.VMEM((B, PB + L), jnp.int32),
            pltpu.VMEM((CH, H), jnp.float32),
            pltpu.VMEM((CH, H), jnp.float32),
            pltpu.VMEM((PH, H), jnp.float32),
            pltpu.VMEM((PH, H), jnp.float32),
            pltpu.VMEM((H,), jnp.float32),
            pltpu.VMEM((H,), jnp.float32),
            pltpu.VMEM((H,), jnp.float32),
            pltpu.SemaphoreType.DMA,
            pltpu.SemaphoreType.DMA,
            pltpu.SemaphoreType.DMA,
            pltpu.SemaphoreType.DMA,
            pltpu.SemaphoreType.DMA,
            pltpu.SemaphoreType.DMA,
        ],
    )
    return fn(ids_flat, tt_flat, word_emb, type_emb, pos_emb)


def kernel(input_ids, token_type_ids, word_emb, type_emb, pos_emb):
    ids_flat = input_ids.reshape(-1).astype(jnp.int32)
    tt_flat = token_type_ids.reshape(-1).astype(jnp.int32)
    out = _emb_call(ids_flat, tt_flat, word_emb, type_emb, pos_emb)
    return out.reshape(B, S, H)


# speedup vs baseline: 1.0379x; 1.0379x over previous
import jax
import jax.numpy as jnp
from jax import lax
from jax.experimental import pallas as pl
from jax.experimental.pallas import tpu as pltpu
from jax.experimental.pallas import tpu_sc as plsc

B, S, H = 4, 2048, 768
V, T = 100000, 2
N = B * S
NC, NS, L = 2, 16, 16
NW = NC * NS
PB = 64
PH = 16
CH = PH * B
NCHUNK = PB // PH
NLG = H // L


def _emb_body(ids_hbm, tt_hbm, word_hbm, type_hbm, pos_hbm, out_hbm,
              idtmp, tttmp, wbuf0, wbuf1, pbuf0, pbuf1,
              t0v, t1v, dvv, isem, psem, gs0, gs1, os0, os1):
    cid = lax.axis_index("c")
    sid = lax.axis_index("s")
    wid = sid * NC + cid
    pbase = wid * PB

    id_handles = []
    for b in range(B):
        id_handles.append(pltpu.async_copy(
            ids_hbm.at[pl.ds(b * S + pbase, PB)], idtmp.at[b], isem))
    aux_handles = []
    for b in range(B):
        aux_handles.append(pltpu.async_copy(
            tt_hbm.at[pl.ds(b * S + pbase, PB)], tttmp.at[b, pl.ds(0, PB)],
            psem))
    aux_handles.append(pltpu.async_copy(type_hbm.at[0], t0v, psem))
    aux_handles.append(pltpu.async_copy(type_hbm.at[1], t1v, psem))
    for cp in id_handles:
        cp.wait()

    wbuf = [wbuf0, wbuf1]
    pbuf = [pbuf0, pbuf1]
    gsem = [gs0, gs1]
    osem = [os0, os1]

    def issue_chunk(h, p):
        hs = []
        for b in range(B):
            hs.append(pltpu.async_copy(
                word_hbm.at[idtmp.at[b, pl.ds(h * PH, PH)]],
                wbuf[p].at[pl.ds(b * PH, PH)], gsem[p]))
        hs.append(pltpu.async_copy(
            pos_hbm.at[pl.ds(pbase + h * PH, PH)], pbuf[p], gsem[p]))
        return hs

    ghandles = [issue_chunk(0, 0), issue_chunk(1, 1)]
    out_handles = [None, None]

    for cp in aux_handles:
        cp.wait()
    for l in range(NLG):
        sl = pl.ds(l * L, L)
        dvv[sl] = t1v[sl] - t0v[sl]

    for h in range(NCHUNK):
        p = h & 1
        if 1 <= h and h + 1 < NCHUNK:
            if out_handles[1 - p] is not None:
                for oh in out_handles[1 - p]:
                    oh.wait()
            ghandles[1 - p] = issue_chunk(h + 1, 1 - p)
        for gh in ghandles[p]:
            gh.wait()

        wb = wbuf[p]
        pb = pbuf[p]
        ttf = [tttmp[b, pl.ds(h * PH, L)].astype(jnp.float32)
               for b in range(B)]

        def l_body(l, carry, wb=wb, pb=pb, ttf=ttf):
            sl = pl.ds(l * L, L)
            t0 = t0v[sl]
            dv = dvv[sl]
            for r in range(PH):
                pp = pb[r, sl] + t0
                for b in range(B):
                    t = b * PH + r
                    wb[t, sl] = wb[t, sl] + pp + ttf[b][r] * dv
            return carry

        lax.fori_loop(0, NLG, l_body, 0)

        out_handles[p] = []
        for b in range(B):
            row0 = b * S + pbase + h * PH
            out_handles[p].append(pltpu.async_copy(
                wb.at[pl.ds(b * PH, PH)],
                out_hbm.at[pl.ds(row0, PH)], osem[p]))

    for hs in out_handles:
        if hs is not None:
            for oh in hs:
                oh.wait()


@jax.jit
def _emb_call(ids_flat, tt_flat, word_emb, type_emb, pos_emb):
    mesh = plsc.VectorSubcoreMesh(core_axis_name="c", subcore_axis_name="s")
    fn = pl.kernel(
        _emb_body,
        out_type=jax.ShapeDtypeStruct((N, H), jnp.float32),
        mesh=mesh,
        compiler_params=pltpu.CompilerParams(use_tc_tiling_on_sc=True),
        scratch_types=[
            pltpu.VMEM((B, PB), jnp.int32),
            pltpu.VMEM((B, PB + L), jnp.int32),
            pltpu.VMEM((CH, H), jnp.float32),
            pltpu.VMEM((CH, H), jnp.float32),
            pltpu.VMEM((PH, H), jnp.float32),
            pltpu.VMEM((PH, H), jnp.float32),
            pltpu.VMEM((H,), jnp.float32),
            pltpu.VMEM((H,), jnp.float32),
            pltpu.VMEM((H,), jnp.float32),
            pltpu.SemaphoreType.DMA,
            pltpu.SemaphoreType.DMA,
            pltpu.SemaphoreType.DMA,
            pltpu.SemaphoreType.DMA,
            pltpu.SemaphoreType.DMA,
            pltpu.SemaphoreType.DMA,
        ],
    )
    return fn(ids_flat, tt_flat, word_emb, type_emb, pos_emb)


def kernel(input_ids, token_type_ids, word_emb, type_emb, pos_emb):
    ids_flat = input_ids.reshape(-1).astype(jnp.int32)
    tt_flat = token_type_ids.reshape(-1).astype(jnp.int32)
    out = _emb_call(ids_flat, tt_flat, word_emb, type_emb, pos_emb)
    return out.reshape(B, S, H)
